# baseline (speedup 1.0000x reference)
import functools
import math

import jax
import jax.numpy as jnp
from jax import lax
from jax.experimental import pallas as pl
from jax.experimental.pallas import tpu as pltpu
from jax.experimental.pallas import tpu_sc as plsc

N = 10000
E = 320000
C = 128
BLK = 1000

NW = 32
ROUNDS = 2
NB = 160
NRANGE = NW * ROUNDS
NPAD = NRANGE * NB
CH = 6400
NCHUNK = E // CH
CAP = 8448
RB = 128

_NEG_INF = float("-inf")
_POS_INF = float("inf")


def _i32v(v):
    return jnp.full((16,), v, jnp.int32)


def _sc_body(x_hbm, src_hbm, dst_hbm, mx_out, mn_out, s_out, c_out,
             dst_chunk, src_chunk, dloc0, src0, dloc1, src1,
             acc_mx, acc_mn, acc_s, acc_c, rows, sem):
    wid = lax.axis_index("s") * 2 + lax.axis_index("c")
    iota = lax.iota(jnp.int32, 16)
    ones = jnp.full((16,), 1.0, jnp.float32)
    lane15 = _i32v(15)
    zero_v = _i32v(0)
    one_v = _i32v(1)
    nb_v = _i32v(NB)
    dump_v = _i32v(CAP - 16) + iota
    lo0_v = _i32v(wid * NB)
    lo1_v = _i32v((NW + wid) * NB)

    def scan_chunk(ci, offs):
        pltpu.sync_copy(dst_hbm.at[pl.ds(ci * CH, CH)], dst_chunk)
        pltpu.sync_copy(src_hbm.at[pl.ds(ci * CH, CH)], src_chunk)

        def scan_vec(t, offs):
            off0, off1 = offs
            ds_ = []
            svs = []
            for t2 in range(4):
                sl = pl.ds(t * 64 + t2 * 16, 16)
                ds_.append(dst_chunk[sl])
                svs.append(src_chunk[sl])
            us0 = []
            ms0 = []
            cs0 = []
            us1 = []
            ms1 = []
            cs1 = []
            for t2 in range(4):
                u0 = ds_[t2] - lo0_v
                m0 = (u0 >= zero_v) & (u0 < nb_v)
                cs0.append(plsc.cumsum(jnp.where(m0, one_v, zero_v)))
                us0.append(u0)
                ms0.append(m0)
                u1 = ds_[t2] - lo1_v
                m1 = (u1 >= zero_v) & (u1 < nb_v)
                cs1.append(plsc.cumsum(jnp.where(m1, one_v, zero_v)))
                us1.append(u1)
                ms1.append(m1)
            ts0 = [c.at[lane15].get(mode="promise_in_bounds") for c in cs0]
            ts1 = [c.at[lane15].get(mode="promise_in_bounds") for c in cs1]
            for t2 in range(4):
                pos0 = jnp.where(ms0[t2], off0 + cs0[t2] - one_v, dump_v)
                plsc.store_scatter(dloc0, [pos0], us0[t2])
                plsc.store_scatter(src0, [pos0], svs[t2])
                off0 = off0 + ts0[t2]
                pos1 = jnp.where(ms1[t2], off1 + cs1[t2] - one_v, dump_v)
                plsc.store_scatter(dloc1, [pos1], us1[t2])
                plsc.store_scatter(src1, [pos1], svs[t2])
                off1 = off1 + ts1[t2]
            return (off0, off1)
        return lax.fori_loop(0, CH // 64, scan_vec, offs)
    off0_v, off1_v = lax.fori_loop(0, NCHUNK, scan_chunk, (zero_v, zero_v))
    n_matches = (off0_v[0], off1_v[0])

    for r in range(ROUNDS):
        lo = (r * NW + wid) * NB
        dloc_list = (dloc0, dloc1)[r]
        src_list = (src0, src1)[r]
        n_match = n_matches[r]

        def init_acc(i, _):
            sl = pl.ds(i * 16, 16)
            acc_mx[sl] = jnp.full((16,), _NEG_INF, jnp.float32)
            acc_mn[sl] = jnp.full((16,), _POS_INF, jnp.float32)
            acc_s[sl] = jnp.zeros((16,), jnp.float32)
            return 0
        lax.fori_loop(0, (NB + 1) * C // 16, init_acc, 0)

        def init_cnt(i, _):
            acc_c[pl.ds(i * 16, 16)] = jnp.zeros((16,), jnp.float32)
            return 0
        lax.fori_loop(0, NB + 1, init_cnt, 0)

        nm_v = _i32v(n_match)
        for t in range(RB // 16 + 1):
            padpos = nm_v + iota + (t * 16)
            plsc.store_scatter(dloc_list, [padpos], nb_v)
            plsc.store_scatter(src_list, [padpos], zero_v)

        n_g = (n_match + RB - 1) // RB

        def gather_chunk(g, _):
            pltpu.async_copy(x_hbm.at[src_list.at[pl.ds(g * RB, RB)]],
                             rows, sem).wait()

            def sub16(sub, _):
                dls = dloc_list[pl.ds(g * RB + sub * 16, 16)]
                for k in range(16):
                    e = sub * 16 + k
                    dl = dls[k]
                    dbase = dl * C
                    for j in range(C // 16):
                        sl_a = pl.ds(dbase + j * 16, 16)
                        row = rows[e, pl.ds(j * 16, 16)]
                        acc_mx[sl_a] = jnp.maximum(acc_mx[sl_a], row)
                        acc_mn[sl_a] = jnp.minimum(acc_mn[sl_a], row)
                        plsc.addupdate(acc_s.at[sl_a], row)
                    plsc.addupdate(acc_c.at[pl.ds(dl * 16, 16)], ones)
                return 0
            lax.fori_loop(0, RB // 16, sub16, 0)
            return 0
        lax.fori_loop(0, n_g, gather_chunk, 0)

        pltpu.sync_copy(acc_mx.at[pl.ds(0, NB * C)],
                        mx_out.at[pl.ds(lo * C, NB * C)])
        pltpu.sync_copy(acc_mn.at[pl.ds(0, NB * C)],
                        mn_out.at[pl.ds(lo * C, NB * C)])
        pltpu.sync_copy(acc_s.at[pl.ds(0, NB * C)],
                        s_out.at[pl.ds(lo * C, NB * C)])
        pltpu.sync_copy(acc_c.at[pl.ds(0, NB * 16)],
                        c_out.at[pl.ds(lo * 16, NB * 16)])


def _sc_aggregate(x, src, dst):
    mesh = plsc.VectorSubcoreMesh(core_axis_name="c", subcore_axis_name="s")
    f = functools.partial(
        pl.kernel, mesh=mesh,
        compiler_params=pltpu.CompilerParams(needs_layout_passes=False),
        out_type=[
            jax.ShapeDtypeStruct((NPAD * C,), jnp.float32),
            jax.ShapeDtypeStruct((NPAD * C,), jnp.float32),
            jax.ShapeDtypeStruct((NPAD * C,), jnp.float32),
            jax.ShapeDtypeStruct((NPAD * 16,), jnp.float32),
        ],
        scratch_types=[
            pltpu.VMEM((CH,), jnp.int32),
            pltpu.VMEM((CH,), jnp.int32),
            pltpu.VMEM((CAP,), jnp.int32),
            pltpu.VMEM((CAP,), jnp.int32),
            pltpu.VMEM((CAP,), jnp.int32),
            pltpu.VMEM((CAP,), jnp.int32),
            pltpu.VMEM(((NB + 1) * C,), jnp.float32),
            pltpu.VMEM(((NB + 1) * C,), jnp.float32),
            pltpu.VMEM(((NB + 1) * C,), jnp.float32),
            pltpu.VMEM(((NB + 1) * 16,), jnp.float32),
            pltpu.VMEM((RB, C), jnp.float32),
            pltpu.SemaphoreType.DMA,
        ],
    )(_sc_body)
    return f(x, src, dst)


def _fusion_body(x_ref, mx_ref, mn_ref, s_ref, cnt_ref,
                 wq_ref, wk_ref, wv_ref, bq_ref, bk_ref, bv_ref,
                 wo_ref, bo_ref, ln_g_ref, ln_b_ref, out_ref):
    x = x_ref[...]
    mx = mx_ref[...]
    mn = mn_ref[...]
    s = s_ref[...]
    cnt = cnt_ref[...]
    mean = s * (1.0 / jnp.maximum(cnt, 1.0))

    mx = jnp.where(mx == _NEG_INF, 0.0, mx)
    mn = jnp.where(mn == _POS_INF, 0.0, mn)

    tokens = (x, mx, mn, s, mean)

    wq = wq_ref[...]
    wk = wk_ref[...]
    wv = wv_ref[...]
    bq = bq_ref[...]
    bk = bk_ref[...]
    bv = bv_ref[...]

    scale = jnp.float32(1.0 / math.sqrt(C))
    q0 = (jnp.dot(x, wq.T, preferred_element_type=jnp.float32) + bq) * scale

    scores = []
    vs = []
    for j, t in enumerate(tokens):
        k_j = jnp.dot(t, wk.T, preferred_element_type=jnp.float32) + bk
        v_j = jnp.dot(t, wv.T, preferred_element_type=jnp.float32) + bv
        s_j = jnp.sum(q0 * k_j, axis=-1)
        if j > 0:
            pad_j = jnp.all(t == 0.0, axis=-1)
            s_j = jnp.where(pad_j, _NEG_INF, s_j)
        scores.append(s_j)
        vs.append(v_j)

    sc = jnp.stack(scores, axis=1)
    m = jnp.max(sc, axis=1, keepdims=True)
    e = jnp.exp(sc - m)
    denom = jnp.sum(e, axis=1, keepdims=True)
    attn = e / denom

    out0 = jnp.zeros_like(x)
    for j in range(5):
        out0 = out0 + attn[:, j][:, None] * vs[j]

    out0 = (jnp.dot(out0, wo_ref[...].T, preferred_element_type=jnp.float32)
            + bo_ref[...])

    mu = jnp.mean(out0, axis=-1, keepdims=True)
    var = jnp.mean((out0 - mu) ** 2, axis=-1, keepdims=True)
    out_ref[...] = ((out0 - mu) * lax.rsqrt(var + 1e-5) * ln_g_ref[...]
                    + ln_b_ref[...])


def _fusion(x, mx, mn, s, cnt, in_proj_w, in_proj_b, out_proj_w, out_proj_b,
            ln_g, ln_b):
    wq = in_proj_w[0:C]
    wk = in_proj_w[C:2 * C]
    wv = in_proj_w[2 * C:3 * C]
    bq = in_proj_b[0:C].reshape(1, C)
    bk = in_proj_b[C:2 * C].reshape(1, C)
    bv = in_proj_b[2 * C:3 * C].reshape(1, C)
    bo = out_proj_b.reshape(1, C)
    g = ln_g.reshape(1, C)
    b = ln_b.reshape(1, C)

    grid = (N // BLK,)
    node_spec = pl.BlockSpec((BLK, C), lambda i: (i, 0))
    cnt_spec = pl.BlockSpec((BLK, 1), lambda i: (i, 0))
    w_spec = pl.BlockSpec((C, C), lambda i: (0, 0))
    b_spec = pl.BlockSpec((1, C), lambda i: (0, 0))

    return pl.pallas_call(
        _fusion_body,
        grid=grid,
        in_specs=[node_spec, node_spec, node_spec, node_spec, cnt_spec,
                  w_spec, w_spec, w_spec, b_spec, b_spec, b_spec,
                  w_spec, b_spec, b_spec, b_spec],
        out_specs=node_spec,
        out_shape=jax.ShapeDtypeStruct((N, C), jnp.float32),
    )(x, mx, mn, s, cnt, wq, wk, wv, bq, bk, bv, out_proj_w, bo, g, b)


def kernel(x, edge_index, in_proj_w, in_proj_b, out_proj_w, out_proj_b,
           ln_g, ln_b):
    src = edge_index[0].astype(jnp.int32)
    dst = edge_index[1].astype(jnp.int32)
    mx_f, mn_f, s_f, c_f = _sc_aggregate(x, src, dst)
    mx = mx_f.reshape(NPAD, C)[:N]
    mn = mn_f.reshape(NPAD, C)[:N]
    s = s_f.reshape(NPAD, C)[:N]
    cnt = c_f.reshape(NPAD, 16)[:N, 0:1]
    return _fusion(x, mx, mn, s, cnt, in_proj_w, in_proj_b,
                   out_proj_w, out_proj_b, ln_g, ln_b)


# speedup vs baseline: 1.1716x; 1.1716x over previous
import functools
import math

import jax
import jax.numpy as jnp
from jax import lax
from jax.experimental import pallas as pl
from jax.experimental.pallas import tpu as pltpu
from jax.experimental.pallas import tpu_sc as plsc

N = 10000
E = 320000
C = 128
BLK = 1000

NW = 32
ROUNDS = 2
NB = 160
NRANGE = NW * ROUNDS
NPAD = NRANGE * NB
CH = 6400
NCHUNK = E // CH
CAP = 6144
RB = 96

_NEG_INF = float("-inf")
_POS_INF = float("inf")


def _i32v(v):
    return jnp.full((16,), v, jnp.int32)


def _sc_body(x_hbm, src_hbm, dst_hbm, mx_out, mn_out, s_out, c_out,
             dst_chunk, src_chunk, dloc0, src0, dloc1, src1,
             acc_mx, acc_mn, acc_s, acc_c, rows_a, rows_b, sem_a, sem_b):
    wid = lax.axis_index("s") * 2 + lax.axis_index("c")
    iota = lax.iota(jnp.int32, 16)
    ones = jnp.full((16,), 1.0, jnp.float32)
    lane15 = _i32v(15)
    zero_v = _i32v(0)
    one_v = _i32v(1)
    nb_v = _i32v(NB)
    dump_v = _i32v(CAP - 16) + iota
    lo0_v = _i32v(wid * NB)
    lo1_v = _i32v((NW + wid) * NB)

    def scan_chunk(ci, offs):
        pltpu.sync_copy(dst_hbm.at[pl.ds(ci * CH, CH)], dst_chunk)
        pltpu.sync_copy(src_hbm.at[pl.ds(ci * CH, CH)], src_chunk)

        def scan_vec(t, offs):
            off0, off1 = offs
            ds_ = []
            svs = []
            for t2 in range(4):
                sl = pl.ds(t * 64 + t2 * 16, 16)
                ds_.append(dst_chunk[sl])
                svs.append(src_chunk[sl])
            us0 = []
            ms0 = []
            cs0 = []
            us1 = []
            ms1 = []
            cs1 = []
            for t2 in range(4):
                u0 = ds_[t2] - lo0_v
                m0 = (u0 >= zero_v) & (u0 < nb_v)
                cs0.append(plsc.cumsum(jnp.where(m0, one_v, zero_v)))
                us0.append(u0)
                ms0.append(m0)
                u1 = ds_[t2] - lo1_v
                m1 = (u1 >= zero_v) & (u1 < nb_v)
                cs1.append(plsc.cumsum(jnp.where(m1, one_v, zero_v)))
                us1.append(u1)
                ms1.append(m1)
            ts0 = [c.at[lane15].get(mode="promise_in_bounds") for c in cs0]
            ts1 = [c.at[lane15].get(mode="promise_in_bounds") for c in cs1]
            for t2 in range(4):
                pos0 = jnp.where(ms0[t2], off0 + cs0[t2] - one_v, dump_v)
                plsc.store_scatter(dloc0, [pos0], us0[t2])
                plsc.store_scatter(src0, [pos0], svs[t2])
                off0 = off0 + ts0[t2]
                pos1 = jnp.where(ms1[t2], off1 + cs1[t2] - one_v, dump_v)
                plsc.store_scatter(dloc1, [pos1], us1[t2])
                plsc.store_scatter(src1, [pos1], svs[t2])
                off1 = off1 + ts1[t2]
            return (off0, off1)
        return lax.fori_loop(0, CH // 64, scan_vec, offs)
    off0_v, off1_v = lax.fori_loop(0, NCHUNK, scan_chunk, (zero_v, zero_v))
    n_matches = (off0_v[0], off1_v[0])

    for r in range(ROUNDS):
        lo = (r * NW + wid) * NB
        dloc_list = (dloc0, dloc1)[r]
        src_list = (src0, src1)[r]
        n_match = n_matches[r]

        def init_acc(i, _):
            sl = pl.ds(i * 16, 16)
            acc_mx[sl] = jnp.full((16,), _NEG_INF, jnp.float32)
            acc_mn[sl] = jnp.full((16,), _POS_INF, jnp.float32)
            acc_s[sl] = jnp.zeros((16,), jnp.float32)
            return 0
        lax.fori_loop(0, (NB + 1) * C // 16, init_acc, 0)

        def init_cnt(i, _):
            acc_c[pl.ds(i * 16, 16)] = jnp.zeros((16,), jnp.float32)
            return 0
        lax.fori_loop(0, NB + 1, init_cnt, 0)

        nm_v = _i32v(n_match)
        for t in range(RB // 16 + 1):
            padpos = nm_v + iota + (t * 16)
            plsc.store_scatter(dloc_list, [padpos], nb_v)
            plsc.store_scatter(src_list, [padpos], zero_v)

        n_g = (n_match + RB - 1) // RB

        def issue(g, buf, s):
            pltpu.async_copy(x_hbm.at[src_list.at[pl.ds(g * RB, RB)]],
                             buf, s)

        def drain(buf, s):
            pltpu.make_async_copy(x_hbm.at[src_list.at[pl.ds(0, RB)]],
                                  buf, s).wait()

        def process(g, buf):
            def sub16(sub, _):
                dls = dloc_list[pl.ds(g * RB + sub * 16, 16)]
                for k in range(16):
                    e = sub * 16 + k
                    dl = dls[k]
                    dbase = dl * C
                    for j in range(C // 16):
                        sl_a = pl.ds(dbase + j * 16, 16)
                        row = buf[e, pl.ds(j * 16, 16)]
                        acc_mx[sl_a] = jnp.maximum(acc_mx[sl_a], row)
                        acc_mn[sl_a] = jnp.minimum(acc_mn[sl_a], row)
                        plsc.addupdate(acc_s.at[sl_a], row)
                    plsc.addupdate(acc_c.at[pl.ds(dl * 16, 16)], ones)
                return 0
            lax.fori_loop(0, RB // 16, sub16, 0)

        @pl.when(n_g > 0)
        def _():
            issue(0, rows_a, sem_a)

        def pair(p, _):
            g0 = p * 2

            @pl.when(g0 + 1 < n_g)
            def _():
                issue(g0 + 1, rows_b, sem_b)
            drain(rows_a, sem_a)
            process(g0, rows_a)

            @pl.when(g0 + 2 < n_g)
            def _():
                issue(g0 + 2, rows_a, sem_a)

            @pl.when(g0 + 1 < n_g)
            def _():
                drain(rows_b, sem_b)
                process(g0 + 1, rows_b)
            return 0
        lax.fori_loop(0, (n_g + 1) // 2, pair, 0)

        pltpu.sync_copy(acc_mx.at[pl.ds(0, NB * C)],
                        mx_out.at[pl.ds(lo * C, NB * C)])
        pltpu.sync_copy(acc_mn.at[pl.ds(0, NB * C)],
                        mn_out.at[pl.ds(lo * C, NB * C)])
        pltpu.sync_copy(acc_s.at[pl.ds(0, NB * C)],
                        s_out.at[pl.ds(lo * C, NB * C)])
        pltpu.sync_copy(acc_c.at[pl.ds(0, NB * 16)],
                        c_out.at[pl.ds(lo * 16, NB * 16)])


def _sc_aggregate(x, src, dst):
    mesh = plsc.VectorSubcoreMesh(core_axis_name="c", subcore_axis_name="s")
    f = functools.partial(
        pl.kernel, mesh=mesh,
        compiler_params=pltpu.CompilerParams(needs_layout_passes=False),
        out_type=[
            jax.ShapeDtypeStruct((NPAD * C,), jnp.float32),
            jax.ShapeDtypeStruct((NPAD * C,), jnp.float32),
            jax.ShapeDtypeStruct((NPAD * C,), jnp.float32),
            jax.ShapeDtypeStruct((NPAD * 16,), jnp.float32),
        ],
        scratch_types=[
            pltpu.VMEM((CH,), jnp.int32),
            pltpu.VMEM((CH,), jnp.int32),
            pltpu.VMEM((CAP,), jnp.int32),
            pltpu.VMEM((CAP,), jnp.int32),
            pltpu.VMEM((CAP,), jnp.int32),
            pltpu.VMEM((CAP,), jnp.int32),
            pltpu.VMEM(((NB + 1) * C,), jnp.float32),
            pltpu.VMEM(((NB + 1) * C,), jnp.float32),
            pltpu.VMEM(((NB + 1) * C,), jnp.float32),
            pltpu.VMEM(((NB + 1) * 16,), jnp.float32),
            pltpu.VMEM((RB, C), jnp.float32),
            pltpu.VMEM((RB, C), jnp.float32),
            pltpu.SemaphoreType.DMA,
            pltpu.SemaphoreType.DMA,
        ],
    )(_sc_body)
    return f(x, src, dst)


def _fusion_body(x_ref, mx_ref, mn_ref, s_ref, cnt_ref,
                 wq_ref, wk_ref, wv_ref, bq_ref, bk_ref, bv_ref,
                 wo_ref, bo_ref, ln_g_ref, ln_b_ref, out_ref):
    x = x_ref[...]
    mx = mx_ref[...]
    mn = mn_ref[...]
    s = s_ref[...]
    cnt = cnt_ref[...]
    mean = s * (1.0 / jnp.maximum(cnt, 1.0))

    mx = jnp.where(mx == _NEG_INF, 0.0, mx)
    mn = jnp.where(mn == _POS_INF, 0.0, mn)

    tokens = (x, mx, mn, s, mean)

    wq = wq_ref[...]
    wk = wk_ref[...]
    wv = wv_ref[...]
    bq = bq_ref[...]
    bk = bk_ref[...]
    bv = bv_ref[...]

    scale = jnp.float32(1.0 / math.sqrt(C))
    q0 = (jnp.dot(x, wq.T, preferred_element_type=jnp.float32) + bq) * scale

    scores = []
    vs = []
    for j, t in enumerate(tokens):
        k_j = jnp.dot(t, wk.T, preferred_element_type=jnp.float32) + bk
        v_j = jnp.dot(t, wv.T, preferred_element_type=jnp.float32) + bv
        s_j = jnp.sum(q0 * k_j, axis=-1)
        if j > 0:
            pad_j = jnp.all(t == 0.0, axis=-1)
            s_j = jnp.where(pad_j, _NEG_INF, s_j)
        scores.append(s_j)
        vs.append(v_j)

    sc = jnp.stack(scores, axis=1)
    m = jnp.max(sc, axis=1, keepdims=True)
    e = jnp.exp(sc - m)
    denom = jnp.sum(e, axis=1, keepdims=True)
    attn = e / denom

    out0 = jnp.zeros_like(x)
    for j in range(5):
        out0 = out0 + attn[:, j][:, None] * vs[j]

    out0 = (jnp.dot(out0, wo_ref[...].T, preferred_element_type=jnp.float32)
            + bo_ref[...])

    mu = jnp.mean(out0, axis=-1, keepdims=True)
    var = jnp.mean((out0 - mu) ** 2, axis=-1, keepdims=True)
    out_ref[...] = ((out0 - mu) * lax.rsqrt(var + 1e-5) * ln_g_ref[...]
                    + ln_b_ref[...])


def _fusion(x, mx, mn, s, cnt, in_proj_w, in_proj_b, out_proj_w, out_proj_b,
            ln_g, ln_b):
    wq = in_proj_w[0:C]
    wk = in_proj_w[C:2 * C]
    wv = in_proj_w[2 * C:3 * C]
    bq = in_proj_b[0:C].reshape(1, C)
    bk = in_proj_b[C:2 * C].reshape(1, C)
    bv = in_proj_b[2 * C:3 * C].reshape(1, C)
    bo = out_proj_b.reshape(1, C)
    g = ln_g.reshape(1, C)
    b = ln_b.reshape(1, C)

    grid = (N // BLK,)
    node_spec = pl.BlockSpec((BLK, C), lambda i: (i, 0))
    cnt_spec = pl.BlockSpec((BLK, 1), lambda i: (i, 0))
    w_spec = pl.BlockSpec((C, C), lambda i: (0, 0))
    b_spec = pl.BlockSpec((1, C), lambda i: (0, 0))

    return pl.pallas_call(
        _fusion_body,
        grid=grid,
        in_specs=[node_spec, node_spec, node_spec, node_spec, cnt_spec,
                  w_spec, w_spec, w_spec, b_spec, b_spec, b_spec,
                  w_spec, b_spec, b_spec, b_spec],
        out_specs=node_spec,
        out_shape=jax.ShapeDtypeStruct((N, C), jnp.float32),
    )(x, mx, mn, s, cnt, wq, wk, wv, bq, bk, bv, out_proj_w, bo, g, b)


def kernel(x, edge_index, in_proj_w, in_proj_b, out_proj_w, out_proj_b,
           ln_g, ln_b):
    src = edge_index[0].astype(jnp.int32)
    dst = edge_index[1].astype(jnp.int32)
    mx_f, mn_f, s_f, c_f = _sc_aggregate(x, src, dst)
    mx = mx_f.reshape(NPAD, C)[:N]
    mn = mn_f.reshape(NPAD, C)[:N]
    s = s_f.reshape(NPAD, C)[:N]
    cnt = c_f.reshape(NPAD, 16)[:N, 0:1]
    return _fusion(x, mx, mn, s, cnt, in_proj_w, in_proj_b,
                   out_proj_w, out_proj_b, ln_g, ln_b)


# speedup vs baseline: 1.3089x; 1.1172x over previous
import functools
import math

import jax
import jax.numpy as jnp
from jax import lax
from jax.experimental import pallas as pl
from jax.experimental.pallas import tpu as pltpu
from jax.experimental.pallas import tpu_sc as plsc

N = 10000
E = 320000
C = 128
BLK = 1000

NW = 32
ROUNDS = 2
NB = 160
NRANGE = NW * ROUNDS
NPAD = NRANGE * NB
CH = 3200
NCHUNK = E // CH
CAP = 6144
RB = 96

_NEG_INF = float("-inf")
_POS_INF = float("inf")


def _i32v(v):
    return jnp.full((16,), v, jnp.int32)


def _sc_body(x_hbm, src_hbm, dst_hbm, mx_out, mn_out, s_out, c_out,
             dst_a, src_a, dst_b, src_b, dloc0, src0, dloc1, src1,
             acc_mx, acc_mn, acc_s, acc_c, rows_a, rows_b,
             sem_a, sem_b, sem_da, sem_sa, sem_db, sem_sb):
    wid = lax.axis_index("s") * 2 + lax.axis_index("c")
    iota = lax.iota(jnp.int32, 16)
    ones = jnp.full((16,), 1.0, jnp.float32)
    lane15 = _i32v(15)
    zero_v = _i32v(0)
    one_v = _i32v(1)
    nb_v = _i32v(NB)
    dump_v = _i32v(CAP - 16) + iota
    lo0_v = _i32v(wid * NB)
    lo1_v = _i32v((NW + wid) * NB)

    def scan_issue(ci, dbuf, sbuf, sd, ss):
        pltpu.async_copy(dst_hbm.at[pl.ds(ci * CH, CH)], dbuf, sd)
        pltpu.async_copy(src_hbm.at[pl.ds(ci * CH, CH)], sbuf, ss)

    def scan_drain(dbuf, sbuf, sd, ss):
        pltpu.make_async_copy(dst_hbm.at[pl.ds(0, CH)], dbuf, sd).wait()
        pltpu.make_async_copy(src_hbm.at[pl.ds(0, CH)], sbuf, ss).wait()

    def scan_process(dst_chunk, src_chunk, offs):
        def scan_vec(t, offs):
            off0, off1 = offs
            ds_ = []
            svs = []
            for t2 in range(4):
                sl = pl.ds(t * 64 + t2 * 16, 16)
                ds_.append(dst_chunk[sl])
                svs.append(src_chunk[sl])
            us0 = []
            ms0 = []
            cs0 = []
            us1 = []
            ms1 = []
            cs1 = []
            for t2 in range(4):
                u0 = ds_[t2] - lo0_v
                m0 = (u0 >= zero_v) & (u0 < nb_v)
                cs0.append(plsc.cumsum(jnp.where(m0, one_v, zero_v)))
                us0.append(u0)
                ms0.append(m0)
                u1 = ds_[t2] - lo1_v
                m1 = (u1 >= zero_v) & (u1 < nb_v)
                cs1.append(plsc.cumsum(jnp.where(m1, one_v, zero_v)))
                us1.append(u1)
                ms1.append(m1)
            ts0 = [c.at[lane15].get(mode="promise_in_bounds") for c in cs0]
            ts1 = [c.at[lane15].get(mode="promise_in_bounds") for c in cs1]
            for t2 in range(4):
                pos0 = jnp.where(ms0[t2], off0 + cs0[t2] - one_v, dump_v)
                plsc.store_scatter(dloc0, [pos0], us0[t2])
                plsc.store_scatter(src0, [pos0], svs[t2])
                off0 = off0 + ts0[t2]
                pos1 = jnp.where(ms1[t2], off1 + cs1[t2] - one_v, dump_v)
                plsc.store_scatter(dloc1, [pos1], us1[t2])
                plsc.store_scatter(src1, [pos1], svs[t2])
                off1 = off1 + ts1[t2]
            return (off0, off1)
        return lax.fori_loop(0, CH // 64, scan_vec, offs)

    scan_issue(0, dst_a, src_a, sem_da, sem_sa)

    def scan_pair(p, offs):
        scan_issue(p * 2 + 1, dst_b, src_b, sem_db, sem_sb)
        scan_drain(dst_a, src_a, sem_da, sem_sa)
        offs = scan_process(dst_a, src_a, offs)

        @pl.when(p * 2 + 2 < NCHUNK)
        def _():
            scan_issue(p * 2 + 2, dst_a, src_a, sem_da, sem_sa)
        scan_drain(dst_b, src_b, sem_db, sem_sb)
        return scan_process(dst_b, src_b, offs)
    off0_v, off1_v = lax.fori_loop(0, NCHUNK // 2, scan_pair,
                                   (zero_v, zero_v))
    n_matches = (off0_v[0], off1_v[0])

    for r in range(ROUNDS):
        lo = (r * NW + wid) * NB
        dloc_list = (dloc0, dloc1)[r]
        src_list = (src0, src1)[r]
        n_match = n_matches[r]

        def init_acc(i, _):
            sl = pl.ds(i * 16, 16)
            acc_mx[sl] = jnp.full((16,), _NEG_INF, jnp.float32)
            acc_mn[sl] = jnp.full((16,), _POS_INF, jnp.float32)
            acc_s[sl] = jnp.zeros((16,), jnp.float32)
            return 0
        lax.fori_loop(0, (NB + 1) * C // 16, init_acc, 0)

        def init_cnt(i, _):
            acc_c[pl.ds(i * 16, 16)] = jnp.zeros((16,), jnp.float32)
            return 0
        lax.fori_loop(0, NB + 1, init_cnt, 0)

        nm_v = _i32v(n_match)
        for t in range(RB // 16 + 1):
            padpos = nm_v + iota + (t * 16)
            plsc.store_scatter(dloc_list, [padpos], nb_v)
            plsc.store_scatter(src_list, [padpos], zero_v)

        n_g = (n_match + RB - 1) // RB

        def issue(g, buf, s):
            pltpu.async_copy(x_hbm.at[src_list.at[pl.ds(g * RB, RB)]],
                             buf, s)

        def drain(buf, s):
            pltpu.make_async_copy(x_hbm.at[src_list.at[pl.ds(0, RB)]],
                                  buf, s).wait()

        def process(g, buf):
            def sub16(sub, _):
                dls = dloc_list[pl.ds(g * RB + sub * 16, 16)]
                for k in range(16):
                    e = sub * 16 + k
                    dl = dls[k]
                    dbase = dl * C
                    for j in range(C // 16):
                        sl_a = pl.ds(dbase + j * 16, 16)
                        row = buf[e, pl.ds(j * 16, 16)]
                        acc_mx[sl_a] = jnp.maximum(acc_mx[sl_a], row)
                        acc_mn[sl_a] = jnp.minimum(acc_mn[sl_a], row)
                        plsc.addupdate(acc_s.at[sl_a], row)
                    plsc.addupdate(acc_c.at[pl.ds(dl * 16, 16)], ones)
                return 0
            lax.fori_loop(0, RB // 16, sub16, 0)

        @pl.when(n_g > 0)
        def _():
            issue(0, rows_a, sem_a)

        def pair(p, _):
            g0 = p * 2

            @pl.when(g0 + 1 < n_g)
            def _():
                issue(g0 + 1, rows_b, sem_b)
            drain(rows_a, sem_a)
            process(g0, rows_a)

            @pl.when(g0 + 2 < n_g)
            def _():
                issue(g0 + 2, rows_a, sem_a)

            @pl.when(g0 + 1 < n_g)
            def _():
                drain(rows_b, sem_b)
                process(g0 + 1, rows_b)
            return 0
        lax.fori_loop(0, (n_g + 1) // 2, pair, 0)

        pltpu.sync_copy(acc_mx.at[pl.ds(0, NB * C)],
                        mx_out.at[pl.ds(lo * C, NB * C)])
        pltpu.sync_copy(acc_mn.at[pl.ds(0, NB * C)],
                        mn_out.at[pl.ds(lo * C, NB * C)])
        pltpu.sync_copy(acc_s.at[pl.ds(0, NB * C)],
                        s_out.at[pl.ds(lo * C, NB * C)])
        pltpu.sync_copy(acc_c.at[pl.ds(0, NB * 16)],
                        c_out.at[pl.ds(lo * 16, NB * 16)])


def _sc_aggregate(x, src, dst):
    mesh = plsc.VectorSubcoreMesh(core_axis_name="c", subcore_axis_name="s")
    f = functools.partial(
        pl.kernel, mesh=mesh,
        compiler_params=pltpu.CompilerParams(needs_layout_passes=False),
        out_type=[
            jax.ShapeDtypeStruct((NPAD * C,), jnp.float32),
            jax.ShapeDtypeStruct((NPAD * C,), jnp.float32),
            jax.ShapeDtypeStruct((NPAD * C,), jnp.float32),
            jax.ShapeDtypeStruct((NPAD * 16,), jnp.float32),
        ],
        scratch_types=[
            pltpu.VMEM((CH,), jnp.int32),
            pltpu.VMEM((CH,), jnp.int32),
            pltpu.VMEM((CH,), jnp.int32),
            pltpu.VMEM((CH,), jnp.int32),
            pltpu.VMEM((CAP,), jnp.int32),
            pltpu.VMEM((CAP,), jnp.int32),
            pltpu.VMEM((CAP,), jnp.int32),
            pltpu.VMEM((CAP,), jnp.int32),
            pltpu.VMEM(((NB + 1) * C,), jnp.float32),
            pltpu.VMEM(((NB + 1) * C,), jnp.float32),
            pltpu.VMEM(((NB + 1) * C,), jnp.float32),
            pltpu.VMEM(((NB + 1) * 16,), jnp.float32),
            pltpu.VMEM((RB, C), jnp.float32),
            pltpu.VMEM((RB, C), jnp.float32),
            pltpu.SemaphoreType.DMA,
            pltpu.SemaphoreType.DMA,
            pltpu.SemaphoreType.DMA,
            pltpu.SemaphoreType.DMA,
            pltpu.SemaphoreType.DMA,
            pltpu.SemaphoreType.DMA,
        ],
    )(_sc_body)
    return f(x, src, dst)


def _fusion_body(x_ref, mx_ref, mn_ref, s_ref, cnt_ref,
                 wq_ref, wk_ref, wv_ref, bq_ref, bk_ref, bv_ref,
                 wo_ref, bo_ref, ln_g_ref, ln_b_ref, out_ref):
    x = x_ref[...]
    mx = mx_ref[...]
    mn = mn_ref[...]
    s = s_ref[...]
    cnt = cnt_ref[...]
    mean = s * (1.0 / jnp.maximum(cnt, 1.0))

    mx = jnp.where(mx == _NEG_INF, 0.0, mx)
    mn = jnp.where(mn == _POS_INF, 0.0, mn)

    tokens = (x, mx, mn, s, mean)

    wq = wq_ref[...]
    wk = wk_ref[...]
    wv = wv_ref[...]
    bq = bq_ref[...]
    bk = bk_ref[...]
    bv = bv_ref[...]

    scale = jnp.float32(1.0 / math.sqrt(C))
    q0 = (jnp.dot(x, wq.T, preferred_element_type=jnp.float32) + bq) * scale

    scores = []
    vs = []
    for j, t in enumerate(tokens):
        k_j = jnp.dot(t, wk.T, preferred_element_type=jnp.float32) + bk
        v_j = jnp.dot(t, wv.T, preferred_element_type=jnp.float32) + bv
        s_j = jnp.sum(q0 * k_j, axis=-1)
        if j > 0:
            pad_j = jnp.all(t == 0.0, axis=-1)
            s_j = jnp.where(pad_j, _NEG_INF, s_j)
        scores.append(s_j)
        vs.append(v_j)

    sc = jnp.stack(scores, axis=1)
    m = jnp.max(sc, axis=1, keepdims=True)
    e = jnp.exp(sc - m)
    denom = jnp.sum(e, axis=1, keepdims=True)
    attn = e / denom

    out0 = jnp.zeros_like(x)
    for j in range(5):
        out0 = out0 + attn[:, j][:, None] * vs[j]

    out0 = (jnp.dot(out0, wo_ref[...].T, preferred_element_type=jnp.float32)
            + bo_ref[...])

    mu = jnp.mean(out0, axis=-1, keepdims=True)
    var = jnp.mean((out0 - mu) ** 2, axis=-1, keepdims=True)
    out_ref[...] = ((out0 - mu) * lax.rsqrt(var + 1e-5) * ln_g_ref[...]
                    + ln_b_ref[...])


def _fusion(x, mx, mn, s, cnt, in_proj_w, in_proj_b, out_proj_w, out_proj_b,
            ln_g, ln_b):
    wq = in_proj_w[0:C]
    wk = in_proj_w[C:2 * C]
    wv = in_proj_w[2 * C:3 * C]
    bq = in_proj_b[0:C].reshape(1, C)
    bk = in_proj_b[C:2 * C].reshape(1, C)
    bv = in_proj_b[2 * C:3 * C].reshape(1, C)
    bo = out_proj_b.reshape(1, C)
    g = ln_g.reshape(1, C)
    b = ln_b.reshape(1, C)

    grid = (N // BLK,)
    node_spec = pl.BlockSpec((BLK, C), lambda i: (i, 0))
    cnt_spec = pl.BlockSpec((BLK, 1), lambda i: (i, 0))
    w_spec = pl.BlockSpec((C, C), lambda i: (0, 0))
    b_spec = pl.BlockSpec((1, C), lambda i: (0, 0))

    return pl.pallas_call(
        _fusion_body,
        grid=grid,
        in_specs=[node_spec, node_spec, node_spec, node_spec, cnt_spec,
                  w_spec, w_spec, w_spec, b_spec, b_spec, b_spec,
                  w_spec, b_spec, b_spec, b_spec],
        out_specs=node_spec,
        out_shape=jax.ShapeDtypeStruct((N, C), jnp.float32),
    )(x, mx, mn, s, cnt, wq, wk, wv, bq, bk, bv, out_proj_w, bo, g, b)


def kernel(x, edge_index, in_proj_w, in_proj_b, out_proj_w, out_proj_b,
           ln_g, ln_b):
    src = edge_index[0].astype(jnp.int32)
    dst = edge_index[1].astype(jnp.int32)
    mx_f, mn_f, s_f, c_f = _sc_aggregate(x, src, dst)
    mx = mx_f.reshape(NPAD, C)[:N]
    mn = mn_f.reshape(NPAD, C)[:N]
    s = s_f.reshape(NPAD, C)[:N]
    cnt = c_f.reshape(NPAD, 16)[:N, 0:1]
    return _fusion(x, mx, mn, s, cnt, in_proj_w, in_proj_b,
                   out_proj_w, out_proj_b, ln_g, ln_b)


# speedup vs baseline: 1.3326x; 1.0181x over previous
import functools
import math

import jax
import jax.numpy as jnp
from jax import lax
from jax.experimental import pallas as pl
from jax.experimental.pallas import tpu as pltpu
from jax.experimental.pallas import tpu_sc as plsc

N = 10000
E = 320000
C = 128
BLK = 1000

NW = 32
ROUNDS = 2
NB = 160
NRANGE = NW * ROUNDS
NPAD = NRANGE * NB
CH = 3200
NCHUNK = E // CH
CAP = 6144
RB = 96

_NEG_INF = float("-inf")
_POS_INF = float("inf")


def _i32v(v):
    return jnp.full((16,), v, jnp.int32)


def _sc_body(x_hbm, src_hbm, dst_hbm, mx_out, mn_out, s_out, c_out,
             dst_a, src_a, dst_b, src_b, dloc0, src0, dloc1, src1,
             acc_mx, acc_mn, acc_s, acc_c, rows_a, rows_b,
             sem_a, sem_b, sem_da, sem_sa, sem_db, sem_sb):
    wid = lax.axis_index("s") * 2 + lax.axis_index("c")
    iota = lax.iota(jnp.int32, 16)
    ones = jnp.full((16,), 1.0, jnp.float32)
    lane15 = _i32v(15)
    zero_v = _i32v(0)
    one_v = _i32v(1)
    nb_v = _i32v(NB)
    dump_v = _i32v(CAP - 16) + iota
    lo0_v = _i32v(wid * NB)
    lo1_v = _i32v((NW + wid) * NB)

    def scan_issue(ci, dbuf, sbuf, sd, ss):
        pltpu.async_copy(dst_hbm.at[pl.ds(ci * CH, CH)], dbuf, sd)
        pltpu.async_copy(src_hbm.at[pl.ds(ci * CH, CH)], sbuf, ss)

    def scan_drain(dbuf, sbuf, sd, ss):
        pltpu.make_async_copy(dst_hbm.at[pl.ds(0, CH)], dbuf, sd).wait()
        pltpu.make_async_copy(src_hbm.at[pl.ds(0, CH)], sbuf, ss).wait()

    def scan_process(dst_chunk, src_chunk, offs):
        def scan_vec(t, offs):
            off0, off1 = offs
            ds_ = []
            svs = []
            for t2 in range(4):
                sl = pl.ds(t * 64 + t2 * 16, 16)
                ds_.append(dst_chunk[sl])
                svs.append(src_chunk[sl])
            us0 = []
            ms0 = []
            cs0 = []
            us1 = []
            ms1 = []
            cs1 = []
            for t2 in range(4):
                u0 = ds_[t2] - lo0_v
                m0 = (u0 >= zero_v) & (u0 < nb_v)
                cs0.append(plsc.cumsum(jnp.where(m0, one_v, zero_v)))
                us0.append(u0)
                ms0.append(m0)
                u1 = ds_[t2] - lo1_v
                m1 = (u1 >= zero_v) & (u1 < nb_v)
                cs1.append(plsc.cumsum(jnp.where(m1, one_v, zero_v)))
                us1.append(u1)
                ms1.append(m1)
            ts0 = [c.at[lane15].get(mode="promise_in_bounds") for c in cs0]
            ts1 = [c.at[lane15].get(mode="promise_in_bounds") for c in cs1]
            for t2 in range(4):
                pos0 = jnp.where(ms0[t2], off0 + cs0[t2] - one_v, dump_v)
                plsc.store_scatter(dloc0, [pos0], us0[t2])
                plsc.store_scatter(src0, [pos0], svs[t2])
                off0 = off0 + ts0[t2]
                pos1 = jnp.where(ms1[t2], off1 + cs1[t2] - one_v, dump_v)
                plsc.store_scatter(dloc1, [pos1], us1[t2])
                plsc.store_scatter(src1, [pos1], svs[t2])
                off1 = off1 + ts1[t2]
            return (off0, off1)
        return lax.fori_loop(0, CH // 64, scan_vec, offs)

    scan_issue(0, dst_a, src_a, sem_da, sem_sa)

    def scan_pair(p, offs):
        scan_issue(p * 2 + 1, dst_b, src_b, sem_db, sem_sb)
        scan_drain(dst_a, src_a, sem_da, sem_sa)
        offs = scan_process(dst_a, src_a, offs)

        @pl.when(p * 2 + 2 < NCHUNK)
        def _():
            scan_issue(p * 2 + 2, dst_a, src_a, sem_da, sem_sa)
        scan_drain(dst_b, src_b, sem_db, sem_sb)
        return scan_process(dst_b, src_b, offs)
    off0_v, off1_v = lax.fori_loop(0, NCHUNK // 2, scan_pair,
                                   (zero_v, zero_v))
    n_matches = (off0_v[0], off1_v[0])

    for r in range(ROUNDS):
        lo = (r * NW + wid) * NB
        dloc_list = (dloc0, dloc1)[r]
        src_list = (src0, src1)[r]
        n_match = n_matches[r]

        def init_acc(i, _):
            sl = pl.ds(i * 16, 16)
            acc_mx[sl] = jnp.full((16,), _NEG_INF, jnp.float32)
            acc_mn[sl] = jnp.full((16,), _POS_INF, jnp.float32)
            acc_s[sl] = jnp.zeros((16,), jnp.float32)
            return 0
        lax.fori_loop(0, (NB + 1) * C // 16, init_acc, 0)

        def init_cnt(i, _):
            acc_c[pl.ds(i * 16, 16)] = jnp.zeros((16,), jnp.float32)
            return 0
        lax.fori_loop(0, NB + 1, init_cnt, 0)

        nm_v = _i32v(n_match)
        for t in range(RB // 16 + 1):
            padpos = nm_v + iota + (t * 16)
            plsc.store_scatter(dloc_list, [padpos], nb_v)
            plsc.store_scatter(src_list, [padpos], zero_v)

        n_g = (n_match + RB - 1) // RB

        def issue(g, buf, s):
            pltpu.async_copy(x_hbm.at[src_list.at[pl.ds(g * RB, RB)]],
                             buf, s)

        def drain(buf, s):
            pltpu.make_async_copy(x_hbm.at[src_list.at[pl.ds(0, RB)]],
                                  buf, s).wait()

        def process(g, buf):
            def sub16(sub, _):
                dls = dloc_list[pl.ds(g * RB + sub * 16, 16)]
                for k in range(16):
                    e = sub * 16 + k
                    dl = dls[k]
                    dbase = dl * C
                    for j in range(C // 16):
                        sl_a = pl.ds(dbase + j * 16, 16)
                        row = buf[e, pl.ds(j * 16, 16)]
                        acc_mx[sl_a] = jnp.maximum(acc_mx[sl_a], row)
                        acc_mn[sl_a] = jnp.minimum(acc_mn[sl_a], row)
                        plsc.addupdate(acc_s.at[sl_a], row)
                    plsc.addupdate(acc_c.at[pl.ds(dl * 16, 16)], ones)
                return 0
            lax.fori_loop(0, RB // 16, sub16, 0)

        @pl.when(n_g > 0)
        def _():
            issue(0, rows_a, sem_a)

        def pair(p, _):
            g0 = p * 2

            @pl.when(g0 + 1 < n_g)
            def _():
                issue(g0 + 1, rows_b, sem_b)
            drain(rows_a, sem_a)
            process(g0, rows_a)

            @pl.when(g0 + 2 < n_g)
            def _():
                issue(g0 + 2, rows_a, sem_a)

            @pl.when(g0 + 1 < n_g)
            def _():
                drain(rows_b, sem_b)
                process(g0 + 1, rows_b)
            return 0
        lax.fori_loop(0, (n_g + 1) // 2, pair, 0)

        pltpu.sync_copy(acc_mx.at[pl.ds(0, NB * C)],
                        mx_out.at[pl.ds(lo * C, NB * C)])
        pltpu.sync_copy(acc_mn.at[pl.ds(0, NB * C)],
                        mn_out.at[pl.ds(lo * C, NB * C)])
        pltpu.sync_copy(acc_s.at[pl.ds(0, NB * C)],
                        s_out.at[pl.ds(lo * C, NB * C)])
        pltpu.sync_copy(acc_c.at[pl.ds(0, NB * 16)],
                        c_out.at[pl.ds(lo * 16, NB * 16)])


def _sc_aggregate(x, src, dst):
    mesh = plsc.VectorSubcoreMesh(core_axis_name="c", subcore_axis_name="s")
    f = functools.partial(
        pl.kernel, mesh=mesh,
        compiler_params=pltpu.CompilerParams(needs_layout_passes=False),
        out_type=[
            jax.ShapeDtypeStruct((NPAD * C,), jnp.float32),
            jax.ShapeDtypeStruct((NPAD * C,), jnp.float32),
            jax.ShapeDtypeStruct((NPAD * C,), jnp.float32),
            jax.ShapeDtypeStruct((NPAD * 16,), jnp.float32),
        ],
        scratch_types=[
            pltpu.VMEM((CH,), jnp.int32),
            pltpu.VMEM((CH,), jnp.int32),
            pltpu.VMEM((CH,), jnp.int32),
            pltpu.VMEM((CH,), jnp.int32),
            pltpu.VMEM((CAP,), jnp.int32),
            pltpu.VMEM((CAP,), jnp.int32),
            pltpu.VMEM((CAP,), jnp.int32),
            pltpu.VMEM((CAP,), jnp.int32),
            pltpu.VMEM(((NB + 1) * C,), jnp.float32),
            pltpu.VMEM(((NB + 1) * C,), jnp.float32),
            pltpu.VMEM(((NB + 1) * C,), jnp.float32),
            pltpu.VMEM(((NB + 1) * 16,), jnp.float32),
            pltpu.VMEM((RB, C), jnp.float32),
            pltpu.VMEM((RB, C), jnp.float32),
            pltpu.SemaphoreType.DMA,
            pltpu.SemaphoreType.DMA,
            pltpu.SemaphoreType.DMA,
            pltpu.SemaphoreType.DMA,
            pltpu.SemaphoreType.DMA,
            pltpu.SemaphoreType.DMA,
        ],
    )(_sc_body)
    return f(x, src, dst)


def _fusion_body(x_ref, mx_ref, mn_ref, s_ref, cnt_ref,
                 wq_ref, wk_ref, wv_ref, bq_ref, bk_ref, bv_ref,
                 wo_ref, bo_ref, ln_g_ref, ln_b_ref, out_ref):
    x = x_ref[...]
    mx = mx_ref[...]
    mn = mn_ref[...]
    s = s_ref[...]
    cnt = cnt_ref[...][:, 0:1]
    mean = s * (1.0 / jnp.maximum(cnt, 1.0))

    mx = jnp.where(mx == _NEG_INF, 0.0, mx)
    mn = jnp.where(mn == _POS_INF, 0.0, mn)

    tokens = (x, mx, mn, s, mean)

    wq = wq_ref[...]
    wk = wk_ref[...]
    wv = wv_ref[...]
    bq = bq_ref[...]
    bk = bk_ref[...]
    bv = bv_ref[...]

    scale = jnp.float32(1.0 / math.sqrt(C))
    q0 = (jnp.dot(x, wq.T, preferred_element_type=jnp.float32) + bq) * scale

    scores = []
    vs = []
    for j, t in enumerate(tokens):
        k_j = jnp.dot(t, wk.T, preferred_element_type=jnp.float32) + bk
        v_j = jnp.dot(t, wv.T, preferred_element_type=jnp.float32) + bv
        s_j = jnp.sum(q0 * k_j, axis=-1)
        if j > 0:
            pad_j = jnp.all(t == 0.0, axis=-1)
            s_j = jnp.where(pad_j, _NEG_INF, s_j)
        scores.append(s_j)
        vs.append(v_j)

    sc = jnp.stack(scores, axis=1)
    m = jnp.max(sc, axis=1, keepdims=True)
    e = jnp.exp(sc - m)
    denom = jnp.sum(e, axis=1, keepdims=True)
    attn = e / denom

    out0 = jnp.zeros_like(x)
    for j in range(5):
        out0 = out0 + attn[:, j][:, None] * vs[j]

    out0 = (jnp.dot(out0, wo_ref[...].T, preferred_element_type=jnp.float32)
            + bo_ref[...])

    mu = jnp.mean(out0, axis=-1, keepdims=True)
    var = jnp.mean((out0 - mu) ** 2, axis=-1, keepdims=True)
    out_ref[...] = ((out0 - mu) * lax.rsqrt(var + 1e-5) * ln_g_ref[...]
                    + ln_b_ref[...])


def _fusion(x, mx, mn, s, cnt, in_proj_w, in_proj_b, out_proj_w, out_proj_b,
            ln_g, ln_b):
    wq = in_proj_w[0:C]
    wk = in_proj_w[C:2 * C]
    wv = in_proj_w[2 * C:3 * C]
    bq = in_proj_b[0:C].reshape(1, C)
    bk = in_proj_b[C:2 * C].reshape(1, C)
    bv = in_proj_b[2 * C:3 * C].reshape(1, C)
    bo = out_proj_b.reshape(1, C)
    g = ln_g.reshape(1, C)
    b = ln_b.reshape(1, C)

    grid = (N // BLK,)
    node_spec = pl.BlockSpec((BLK, C), lambda i: (i, 0))
    cnt_spec = pl.BlockSpec((BLK, 16), lambda i: (i, 0))
    w_spec = pl.BlockSpec((C, C), lambda i: (0, 0))
    b_spec = pl.BlockSpec((1, C), lambda i: (0, 0))

    return pl.pallas_call(
        _fusion_body,
        grid=grid,
        in_specs=[node_spec, node_spec, node_spec, node_spec, cnt_spec,
                  w_spec, w_spec, w_spec, b_spec, b_spec, b_spec,
                  w_spec, b_spec, b_spec, b_spec],
        out_specs=node_spec,
        out_shape=jax.ShapeDtypeStruct((N, C), jnp.float32),
    )(x, mx, mn, s, cnt, wq, wk, wv, bq, bk, bv, out_proj_w, bo, g, b)


def kernel(x, edge_index, in_proj_w, in_proj_b, out_proj_w, out_proj_b,
           ln_g, ln_b):
    src = edge_index[0].astype(jnp.int32)
    dst = edge_index[1].astype(jnp.int32)
    mx_f, mn_f, s_f, c_f = _sc_aggregate(x, src, dst)
    mx = mx_f.reshape(NPAD, C)
    mn = mn_f.reshape(NPAD, C)
    s = s_f.reshape(NPAD, C)
    cnt = c_f.reshape(NPAD, 16)
    return _fusion(x, mx, mn, s, cnt, in_proj_w, in_proj_b,
                   out_proj_w, out_proj_b, ln_g, ln_b)
